# baseline (speedup 1.0000x reference)
import jax
import jax.numpy as jnp
from jax.experimental import pallas as pl
from jax.experimental.pallas import tpu as pltpu

D_MODEL = 1024
D_SPACE = 64
S = 2048
N_TOTAL = 16384
N_OUT = 20480
TN = 2048
NUM_J = N_OUT // TN

_HTAB = (0, 1, 2, 3, 3, 4, 5, 6, 7, 7)
_NROW = (0, 0, 2048, 8192, 10240, 4096, 4096, 6144, 12288, 14336)


def _ln_into(scr, k, t, g_ref, b_ref):
    g = g_ref[:, k * D_SPACE:(k + 1) * D_SPACE]
    b = b_ref[:, k * D_SPACE:(k + 1) * D_SPACE]
    m = jnp.mean(t, axis=-1, keepdims=True)
    v = jnp.mean((t - m) ** 2, axis=-1, keepdims=True)
    scr[k] = ((t - m) * jax.lax.rsqrt(v + 1e-5) * g + b).astype(jnp.bfloat16)


def _body(tab_ref, x_ref, ca_ref, ck_ref, ne_ref, Wx_ref, bx_ref, Wr_ref,
          br_ref, Wkn_ref, bkn_ref, g_ref, beta_ref, out_ref, h_scr, ne_scr):
    j = pl.program_id(0)

    @pl.when(j == 0)
    def _prologue():
        px = jnp.dot(x_ref[...], Wx_ref[...],
                     preferred_element_type=jnp.float32) + bx_ref[...]
        pr = jnp.dot(ca_ref[...], Wr_ref[...],
                     preferred_element_type=jnp.float32) + br_ref[...]
        pk = jnp.dot(ck_ref[...], Wkn_ref[...],
                     preferred_element_type=jnp.float32) + bkn_ref[...]
        for k in range(4):
            _ln_into(h_scr, k, px[:, k * D_SPACE:(k + 1) * D_SPACE],
                     g_ref, beta_ref)
        for k in range(3):
            _ln_into(h_scr, 4 + k, pr[:, k * D_SPACE:(k + 1) * D_SPACE],
                     g_ref, beta_ref)
        _ln_into(h_scr, 7, pk, g_ref, beta_ref)
        e = ne_ref[...]
        inv = 1.0 / jnp.maximum(
            jnp.sqrt(jnp.sum(e * e, axis=-1, keepdims=True)), 1e-12)
        ne_scr[...] = (e * inv).astype(jnp.bfloat16)

    @pl.when(j > 0)
    def _main():
        jj = j - 1
        row = pl.multiple_of(tab_ref[0, jj], TN)
        en = ne_scr[pl.ds(row, TN), :]
        h = h_scr[tab_ref[1, jj]]
        out_ref[...] = jax.lax.dot_general(
            h, en, (((1,), (1,)), ((), ())),
            preferred_element_type=jnp.float32)


def kernel(x, ctx_attn, ctx_know, neuron_emb, W_feat, b_feat, W_know, b_know,
           W_rQ, b_rQ, W_rK, b_rK, W_rV, b_rV, W_rKn, b_rKn,
           g_fqkQ, beta_fqkQ, g_fqkK, beta_fqkK, g_fv, beta_fv,
           g_fkn, beta_fkn, g_rQ, beta_rQ, g_rK, beta_rK,
           g_rV, beta_rV, g_rKn, beta_rKn):
    B = x.shape[0]
    x2 = x.reshape(B * S, D_MODEL).astype(jnp.bfloat16)
    ca = ctx_attn.reshape(B * S, -1).astype(jnp.bfloat16)
    ck = ctx_know.reshape(B * S, -1).astype(jnp.bfloat16)

    Wx = jnp.concatenate([W_feat, W_know], axis=1)
    bx = jnp.concatenate([b_feat, b_know])[None, :]
    Wr = jnp.concatenate([W_rQ, W_rK, W_rV], axis=1)
    br = jnp.concatenate([b_rQ, b_rK, b_rV])[None, :]
    bkn = b_rKn[None, :]
    g = jnp.concatenate([g_fqkQ, g_fqkK, g_fv, g_fkn,
                         g_rQ, g_rK, g_rV, g_rKn])[None, :]
    beta = jnp.concatenate([beta_fqkQ, beta_fqkK, beta_fv, beta_fkn,
                            beta_rQ, beta_rK, beta_rV, beta_rKn])[None, :]

    Wx = Wx.astype(jnp.bfloat16)
    Wr = Wr.astype(jnp.bfloat16)
    Wkn = W_rKn.astype(jnp.bfloat16)

    tab = jnp.asarray([_NROW, _HTAB], dtype=jnp.int32)
    full = lambda a: pl.BlockSpec(a.shape, lambda j, t: (0,) * a.ndim)

    grid_spec = pltpu.PrefetchScalarGridSpec(
        num_scalar_prefetch=1,
        grid=(NUM_J + 1,),
        in_specs=[
            full(x2), full(ca), full(ck), full(neuron_emb),
            full(Wx), full(bx), full(Wr), full(br),
            full(Wkn), full(bkn), full(g), full(beta),
        ],
        out_specs=pl.BlockSpec((B * S, TN),
                               lambda j, t: (0, jnp.maximum(j - 1, 0))),
        scratch_shapes=[pltpu.VMEM((8, B * S, D_SPACE), jnp.bfloat16),
                        pltpu.VMEM((N_TOTAL, D_SPACE), jnp.bfloat16)],
    )

    out = pl.pallas_call(
        _body,
        grid_spec=grid_spec,
        out_shape=jax.ShapeDtypeStruct((B * S, N_OUT), jnp.float32),
    )(tab, x2, ca, ck, neuron_emb, Wx, bx, Wr, br, Wkn, bkn, g, beta)

    return out.reshape(B, S, N_OUT)


# speedup vs baseline: 1.0915x; 1.0915x over previous
import jax
import jax.numpy as jnp
from jax.experimental import pallas as pl
from jax.experimental.pallas import tpu as pltpu

D_MODEL = 1024
D_SPACE = 64
S = 2048
N_OUT = 20480
TN = 1024
NUM_J = N_OUT // TN
XK = D_MODEL // 2

_NTAB = (4, 5, 4, 5, 6, 7, 12, 13, 14, 15, 0, 1, 0, 1, 2, 3, 8, 9, 10, 11)
_OTAB = (10, 11, 12, 13, 14, 15, 16, 17, 18, 19, 0, 1, 2, 3, 4, 5, 6, 7, 8, 9)
_HTAB = (4, 4, 5, 5, 6, 6, 7, 7, 7, 7, 0, 0, 1, 1, 2, 2, 3, 3, 3, 3)


def _ln_into(scr, k, t, g_ref, b_ref):
    g = g_ref[:, k * D_SPACE:(k + 1) * D_SPACE]
    b = b_ref[:, k * D_SPACE:(k + 1) * D_SPACE]
    m = jnp.mean(t, axis=-1, keepdims=True)
    v = jnp.mean((t - m) ** 2, axis=-1, keepdims=True)
    scr[k] = ((t - m) * jax.lax.rsqrt(v + 1e-5) * g + b).astype(jnp.bfloat16)


def _body(tab_ref, x_ref, ca_ref, ck_ref, ne_ref, Wx_ref, bx_ref, Wr_ref,
          br_ref, Wkn_ref, bkn_ref, g_ref, beta_ref, out_ref, h_scr, px_scr):
    s = pl.program_id(0)

    @pl.when(s == 0)
    def _ctx_prologue():
        pr = jnp.dot(ca_ref[...], Wr_ref[...],
                     preferred_element_type=jnp.float32) + br_ref[...]
        pk = jnp.dot(ck_ref[...], Wkn_ref[...],
                     preferred_element_type=jnp.float32) + bkn_ref[...]
        for k in range(3):
            _ln_into(h_scr, 4 + k, pr[:, k * D_SPACE:(k + 1) * D_SPACE],
                     g_ref, beta_ref)
        _ln_into(h_scr, 7, pk, g_ref, beta_ref)
        px_scr[...] = jnp.dot(x_ref[...], Wx_ref[0],
                              preferred_element_type=jnp.float32)

    @pl.when(s == 1)
    def _x_prologue():
        px = px_scr[...] + jnp.dot(x_ref[...], Wx_ref[1],
                                   preferred_element_type=jnp.float32)
        px = px + bx_ref[...]
        for k in range(4):
            _ln_into(h_scr, k, px[:, k * D_SPACE:(k + 1) * D_SPACE],
                     g_ref, beta_ref)

    e = ne_ref[...]
    inv = 1.0 / jnp.maximum(
        jnp.sqrt(jnp.sum(e * e, axis=-1, keepdims=True)), 1e-12)
    en = (e * inv).astype(jnp.bfloat16)
    h = h_scr[tab_ref[2, s]]
    out_ref[...] = jax.lax.dot_general(
        h, en, (((1,), (1,)), ((), ())), preferred_element_type=jnp.float32)


def kernel(x, ctx_attn, ctx_know, neuron_emb, W_feat, b_feat, W_know, b_know,
           W_rQ, b_rQ, W_rK, b_rK, W_rV, b_rV, W_rKn, b_rKn,
           g_fqkQ, beta_fqkQ, g_fqkK, beta_fqkK, g_fv, beta_fv,
           g_fkn, beta_fkn, g_rQ, beta_rQ, g_rK, beta_rK,
           g_rV, beta_rV, g_rKn, beta_rKn):
    B = x.shape[0]
    x2 = x.reshape(B * S, D_MODEL).astype(jnp.bfloat16)
    ca = ctx_attn.reshape(B * S, -1).astype(jnp.bfloat16)
    ck = ctx_know.reshape(B * S, -1).astype(jnp.bfloat16)

    Wx = jnp.concatenate([W_feat, W_know], axis=1)
    Wxs = Wx.astype(jnp.bfloat16).reshape(2, XK, 256)
    bx = jnp.concatenate([b_feat, b_know])[None, :]
    Wr = jnp.concatenate([W_rQ, W_rK, W_rV], axis=1).astype(jnp.bfloat16)
    br = jnp.concatenate([b_rQ, b_rK, b_rV])[None, :]
    Wkn = W_rKn.astype(jnp.bfloat16)
    bkn = b_rKn[None, :]
    g = jnp.concatenate([g_fqkQ, g_fqkK, g_fv, g_fkn,
                         g_rQ, g_rK, g_rV, g_rKn])[None, :]
    beta = jnp.concatenate([beta_fqkQ, beta_fqkK, beta_fv, beta_fkn,
                            beta_rQ, beta_rK, beta_rV, beta_rKn])[None, :]

    tab = jnp.asarray([_NTAB, _OTAB, _HTAB], dtype=jnp.int32)
    full = lambda a: pl.BlockSpec(a.shape, lambda s, t: (0,) * a.ndim)

    grid_spec = pltpu.PrefetchScalarGridSpec(
        num_scalar_prefetch=1,
        grid=(NUM_J,),
        in_specs=[
            pl.BlockSpec((B * S, XK), lambda s, t: (0, jnp.minimum(s, 1))),
            full(ca), full(ck),
            pl.BlockSpec((TN, D_SPACE), lambda s, t: (t[0, s], 0)),
            full(Wxs), full(bx), full(Wr), full(br),
            full(Wkn), full(bkn), full(g), full(beta),
        ],
        out_specs=pl.BlockSpec((B * S, TN), lambda s, t: (0, t[1, s])),
        scratch_shapes=[pltpu.VMEM((8, B * S, D_SPACE), jnp.bfloat16),
                        pltpu.VMEM((B * S, 256), jnp.float32)],
    )

    out = pl.pallas_call(
        _body,
        grid_spec=grid_spec,
        out_shape=jax.ShapeDtypeStruct((B * S, N_OUT), jnp.float32),
    )(tab, x2, ca, ck, neuron_emb, Wxs, bx, Wr, br, Wkn, bkn, g, beta)

    return out.reshape(B, S, N_OUT)


# speedup vs baseline: 1.1138x; 1.0204x over previous
import jax
import jax.numpy as jnp
from jax.experimental import pallas as pl
from jax.experimental.pallas import tpu as pltpu

D_MODEL = 1024
D_SPACE = 64
S = 2048
N_OUT = 20480
TN = 1024
NUM_J = N_OUT // TN
XK = D_MODEL // 2

_NTAB = (4, 5, 4, 5, 6, 7, 12, 13, 14, 15, 0, 1, 0, 1, 2, 3, 8, 9, 10, 11)
_OTAB = (10, 11, 12, 13, 14, 15, 16, 17, 18, 19, 0, 1, 2, 3, 4, 5, 6, 7, 8, 9)
_HTAB = (4, 4, 5, 5, 6, 6, 7, 7, 7, 7, 0, 0, 1, 1, 2, 2, 3, 3, 3, 3)


def _ln_into(scr, k, t, g_ref, b_ref):
    g = g_ref[:, k * D_SPACE:(k + 1) * D_SPACE]
    b = b_ref[:, k * D_SPACE:(k + 1) * D_SPACE]
    m = jnp.mean(t, axis=-1, keepdims=True)
    v = jnp.mean((t - m) ** 2, axis=-1, keepdims=True)
    scr[k] = ((t - m) * jax.lax.rsqrt(v + 1e-5) * g + b).astype(jnp.bfloat16)


def _body(tab_ref, x_ref, ca_ref, ck_ref, ne_ref, Wx_ref, bx_ref, Wr_ref,
          br_ref, Wkn_ref, bkn_ref, g_ref, beta_ref, out_ref, h_scr, px_scr):
    s = pl.program_id(0)

    @pl.when(s == 0)
    def _ctx_prologue():
        pr = jnp.dot(ca_ref[...], Wr_ref[...],
                     preferred_element_type=jnp.float32) + br_ref[...]
        pk = jnp.dot(ck_ref[...], Wkn_ref[...],
                     preferred_element_type=jnp.float32) + bkn_ref[...]
        for k in range(3):
            _ln_into(h_scr, 4 + k, pr[:, k * D_SPACE:(k + 1) * D_SPACE],
                     g_ref, beta_ref)
        _ln_into(h_scr, 7, pk, g_ref, beta_ref)

    @pl.when(s == 1)
    def _x_prologue_a():
        px_scr[...] = jnp.dot(x_ref[...], Wx_ref[0],
                              preferred_element_type=jnp.float32)

    @pl.when(s == 2)
    def _x_prologue_b():
        px = px_scr[...] + jnp.dot(x_ref[...], Wx_ref[1],
                                   preferred_element_type=jnp.float32)
        px = px + bx_ref[...]
        for k in range(4):
            _ln_into(h_scr, k, px[:, k * D_SPACE:(k + 1) * D_SPACE],
                     g_ref, beta_ref)

    e = ne_ref[...].astype(jnp.float32)
    inv = 1.0 / jnp.maximum(
        jnp.sqrt(jnp.sum(e * e, axis=-1, keepdims=True)), 1e-12)
    en = (e * inv).astype(jnp.bfloat16)
    h = h_scr[tab_ref[2, s]]
    out_ref[...] = jax.lax.dot_general(
        h, en, (((1,), (1,)), ((), ())), preferred_element_type=jnp.float32)


def kernel(x, ctx_attn, ctx_know, neuron_emb, W_feat, b_feat, W_know, b_know,
           W_rQ, b_rQ, W_rK, b_rK, W_rV, b_rV, W_rKn, b_rKn,
           g_fqkQ, beta_fqkQ, g_fqkK, beta_fqkK, g_fv, beta_fv,
           g_fkn, beta_fkn, g_rQ, beta_rQ, g_rK, beta_rK,
           g_rV, beta_rV, g_rKn, beta_rKn):
    B = x.shape[0]
    x2 = x.reshape(B * S, D_MODEL).astype(jnp.bfloat16)
    ca = ctx_attn.reshape(B * S, -1).astype(jnp.bfloat16)
    ck = ctx_know.reshape(B * S, -1).astype(jnp.bfloat16)

    Wx = jnp.concatenate([W_feat, W_know], axis=1)
    Wxs = Wx.astype(jnp.bfloat16).reshape(2, XK, 256)
    bx = jnp.concatenate([b_feat, b_know])[None, :]
    Wr = jnp.concatenate([W_rQ, W_rK, W_rV], axis=1).astype(jnp.bfloat16)
    br = jnp.concatenate([b_rQ, b_rK, b_rV])[None, :]
    Wkn = W_rKn.astype(jnp.bfloat16)
    bkn = b_rKn[None, :]
    g = jnp.concatenate([g_fqkQ, g_fqkK, g_fv, g_fkn,
                         g_rQ, g_rK, g_rV, g_rKn])[None, :]
    beta = jnp.concatenate([beta_fqkQ, beta_fqkK, beta_fv, beta_fkn,
                            beta_rQ, beta_rK, beta_rV, beta_rKn])[None, :]

    tab = jnp.asarray([_NTAB, _OTAB, _HTAB], dtype=jnp.int32)
    full = lambda a: pl.BlockSpec(a.shape, lambda s, t: (0,) * a.ndim)

    grid_spec = pltpu.PrefetchScalarGridSpec(
        num_scalar_prefetch=1,
        grid=(NUM_J,),
        in_specs=[
            pl.BlockSpec((B * S, XK),
                         lambda s, t: (0, jnp.clip(s - 1, 0, 1))),
            full(ca), full(ck),
            pl.BlockSpec((TN, D_SPACE), lambda s, t: (t[0, s], 0)),
            full(Wxs), full(bx), full(Wr), full(br),
            full(Wkn), full(bkn), full(g), full(beta),
        ],
        out_specs=pl.BlockSpec((B * S, TN), lambda s, t: (0, t[1, s])),
        scratch_shapes=[pltpu.VMEM((8, B * S, D_SPACE), jnp.bfloat16),
                        pltpu.VMEM((B * S, 256), jnp.float32)],
    )

    out = pl.pallas_call(
        _body,
        grid_spec=grid_spec,
        out_shape=jax.ShapeDtypeStruct((B * S, N_OUT), jnp.float32),
    )(tab, x2, ca, ck, neuron_emb.astype(jnp.bfloat16),
      Wxs, bx, Wr, br, Wkn, bkn, g, beta)

    return out.reshape(B, S, N_OUT)
